# baseline (speedup 1.0000x reference)
import functools

import jax
import jax.numpy as jnp
from jax import lax
from jax.experimental import pallas as pl
from jax.experimental.pallas import tpu as pltpu
from jax.experimental.pallas import tpu_sc as plsc

N_KENN_LAYERS = 3
NC = 2
NS = 16
NW = NC * NS
L = 16

RPW = 384
NCH = RPW // 128


def _mlp_body(x_ref, w1_ref, b1_ref, w2_ref, b2_ref, o0_ref, o1_ref, o2_ref):
    h = jnp.maximum(
        jnp.dot(x_ref[...], w1_ref[...], preferred_element_type=jnp.float32)
        + b1_ref[...],
        0.0,
    )
    res = jax.lax.dot_general(
        w2_ref[...], h, (((0,), (1,)), ((), ())),
        preferred_element_type=jnp.float32,
    ) + b2_ref[...]
    o0_ref[...] = res[0]
    o1_ref[...] = res[1]
    o2_ref[...] = res[2]


def _mlp_cols(features, W1, b1, W2, b2):
    n, d = features.shape
    blk = 1024
    grid = (n + blk - 1) // blk
    col = jax.ShapeDtypeStruct((n,), jnp.float32)
    return pl.pallas_call(
        _mlp_body,
        grid=(grid,),
        in_specs=[
            pl.BlockSpec((blk, d), lambda i: (i, 0)),
            pl.BlockSpec((d, W1.shape[1]), lambda i: (0, 0)),
            pl.BlockSpec((1, W1.shape[1]), lambda i: (0, 0)),
            pl.BlockSpec((W1.shape[1], W2.shape[1]), lambda i: (0, 0)),
            pl.BlockSpec((W2.shape[1], 1), lambda i: (0, 0)),
        ],
        out_specs=[
            pl.BlockSpec((blk,), lambda i: (i,)),
            pl.BlockSpec((blk,), lambda i: (i,)),
            pl.BlockSpec((blk,), lambda i: (i,)),
        ],
        out_shape=[col, col, col],
        compiler_params=pltpu.CompilerParams(
            dimension_semantics=("arbitrary",),
        ),
    )(features, W1, b1.reshape(1, -1), W2, b2.reshape(-1, 1))


def _make_winner_kernel(E, M, N):
    EPW = E // NW
    mesh = plsc.VectorSubcoreMesh(
        core_axis_name="c", subcore_axis_name="s", num_cores=NC, num_subcores=NS
    )

    @functools.partial(
        pl.kernel,
        out_type=jax.ShapeDtypeStruct((NW * N,), jnp.int32),
        mesh=mesh,
        compiler_params=pltpu.CompilerParams(needs_layout_passes=False),
        scratch_types=[
            pltpu.VMEM((EPW,), jnp.int32),
            pltpu.VMEM((N,), jnp.int32),
            pltpu.VMEM((EPW + L,), jnp.int32),
            pltpu.VMEM((EPW + L,), jnp.int32),
            pltpu.SemaphoreType.DMA,
        ],
    )
    def winner_kernel(idxxz_hbm, win_hbm, idx_v, win_v, rowc, evc, sem):
        wid = lax.axis_index("s") * NC + lax.axis_index("c")
        pltpu.async_copy(idxxz_hbm.at[pl.ds(wid * EPW, EPW)], idx_v, sem).wait()

        neg1 = jnp.full((L,), -1, jnp.int32)

        def init_body(i, _):
            win_v[pl.ds(i * L, L)] = neg1
            return 0

        lax.fori_loop(0, N // L, init_body, 0)

        lanes = lax.iota(jnp.int32, L)
        ebase = wid * EPW

        def compact_body(i, off):
            v = idx_v[pl.ds(i * L, L)]
            msk = v >= M
            evec = (ebase + i * L) + lanes
            plsc.store_compressed(rowc.at[pl.ds(off, L)], v - M, mask=msk)
            plsc.store_compressed(evc.at[pl.ds(off, L)], evec, mask=msk)
            return off + plsc.all_reduce_population_count(msk)[0]

        total = lax.fori_loop(0, EPW // L, compact_body, 0)

        def settle_body(j, _):
            rows = rowc[pl.ds(j * L, L)]
            evec = evc[pl.ds(j * L, L)]
            lanemask = (j * L + lanes) < total

            def w_cond(b):
                return jnp.max(b) > 0

            def w_body(b):
                plsc.store_scatter(win_v, [rows], evec, mask=b != 0)
                g = plsc.load_gather(win_v, [rows], mask=lanemask)
                return (lanemask & (g < evec)).astype(jnp.int32)

            lax.while_loop(w_cond, w_body, lanemask.astype(jnp.int32))
            return 0

        lax.fori_loop(0, (total + L - 1) // L, settle_body, 0)
        pltpu.sync_copy(win_v, win_hbm.at[pl.ds(wid * N, N)])

    return winner_kernel


def _make_b1_kernel(E, M, N):
    mesh = plsc.VectorSubcoreMesh(
        core_axis_name="c", subcore_axis_name="s", num_cores=NC, num_subcores=NS
    )
    flat = jax.ShapeDtypeStruct((NW * RPW,), jnp.int32)

    @functools.partial(
        pl.kernel,
        out_type=(flat, flat, flat),
        mesh=mesh,
        compiler_params=pltpu.CompilerParams(needs_layout_passes=False),
        scratch_types=[
            pltpu.VMEM((NW * RPW,), jnp.int32),
            pltpu.VMEM((RPW,), jnp.int32),
            pltpu.VMEM((RPW,), jnp.int32),
            pltpu.VMEM((RPW,), jnp.int32),
            pltpu.VMEM((RPW,), jnp.int32),
            pltpu.SemaphoreType.DMA,
        ],
    )
    def b1_kernel(win_hbm, ixy_hbm, iyz_hbm,
                  winv_hbm, gxy_hbm, gyz_hbm,
                  winb, winv, ebuf, gxy, gyz, sem):
        wid = lax.axis_index("s") * NC + lax.axis_index("c")
        base = jnp.minimum(wid * RPW, N - RPW)
        obase = wid * RPW

        ds0 = [
            pltpu.async_copy(win_hbm.at[pl.ds(t * N + base, RPW)],
                             winb.at[pl.ds(t * RPW, RPW)], sem)
            for t in range(NW)
        ]
        for d in ds0:
            d.wait()

        def red_body(i, _):
            acc = jnp.full((L,), -1, jnp.int32)
            for t in range(NW):
                acc = jnp.maximum(acc, winb[pl.ds(t * RPW + i * L, L)])
            winv[pl.ds(i * L, L)] = acc
            ebuf[pl.ds(i * L, L)] = jnp.maximum(acc, 0)
            return 0

        lax.fori_loop(0, RPW // L, red_body, 0)

        ds1 = []
        for j in range(NCH):
            sl = pl.ds(j * 128, 128)
            ds1.append(pltpu.async_copy(ixy_hbm.at[ebuf.at[sl]], gxy.at[sl], sem))
            ds1.append(pltpu.async_copy(iyz_hbm.at[ebuf.at[sl]], gyz.at[sl], sem))
        for d in ds1:
            d.wait()

        osl = pl.ds(obase, RPW)
        pltpu.sync_copy(winv, winv_hbm.at[osl])
        pltpu.sync_copy(gxy, gxy_hbm.at[osl])
        pltpu.sync_copy(gyz, gyz_hbm.at[osl])

    return b1_kernel


def _make_b2_kernel(E, M, N):
    mesh = plsc.VectorSubcoreMesh(
        core_axis_name="c", subcore_axis_name="s", num_cores=NC, num_subcores=NS
    )

    @functools.partial(
        pl.kernel,
        out_type=(
            jax.ShapeDtypeStruct((N, 3), jnp.float32),
            jax.ShapeDtypeStruct((N, 3), jnp.float32),
        ),
        mesh=mesh,
        compiler_params=pltpu.CompilerParams(needs_layout_passes=False),
        scratch_types=[
            pltpu.VMEM((RPW,), jnp.int32),
            pltpu.VMEM((RPW,), jnp.int32),
            pltpu.VMEM((RPW,), jnp.int32),
            pltpu.VMEM((RPW,), jnp.int32),
            pltpu.VMEM((RPW,), jnp.int32),
            pltpu.VMEM((RPW,), jnp.int32),
            pltpu.VMEM((RPW,), jnp.int32),
            pltpu.VMEM((RPW,), jnp.float32),
            pltpu.VMEM((RPW,), jnp.float32),
            pltpu.VMEM((RPW,), jnp.float32),
            pltpu.VMEM((RPW,), jnp.float32),
            pltpu.VMEM((RPW,), jnp.float32),
            pltpu.VMEM((RPW,), jnp.float32),
            pltpu.VMEM((RPW,), jnp.float32),
            pltpu.VMEM((RPW,), jnp.float32),
            pltpu.VMEM((RPW,), jnp.float32),
            pltpu.VMEM((RPW,), jnp.float32),
            pltpu.VMEM((RPW,), jnp.float32),
            pltpu.VMEM((RPW,), jnp.float32),
            pltpu.VMEM((RPW,), jnp.float32),
            pltpu.VMEM((RPW,), jnp.float32),
            pltpu.VMEM((RPW,), jnp.float32),
            pltpu.VMEM((RPW, 3), jnp.float32),
            pltpu.VMEM((RPW, 3), jnp.float32),
            pltpu.VMEM((L,), jnp.float32),
            pltpu.SemaphoreType.DMA,
            pltpu.SemaphoreType.DMA,
        ],
    )
    def b2_kernel(winv_hbm, gxy_hbm, gyz_hbm,
                  w0_hbm, w1_hbm, w2_hbm, a0_hbm, a1_hbm, a2_hbm, wsp_hbm,
                  out_hbm, soft_hbm,
                  winv, rxy, ryz, xywi, xyai, yzwi, yzai,
                  xyw0, xyw1, xyw2, xya0, xya1, xya2,
                  yzw0, yzw1, yzw2, yza0, yza1, yza2,
                  xz0, xz1, xz2, obf, sbf, wv, sem1, sem2):
        wid = lax.axis_index("s") * NC + lax.axis_index("c")
        base = jnp.minimum(wid * RPW, N - RPW)
        obase = wid * RPW
        osl = pl.ds(obase, RPW)
        wcols = (w0_hbm, w1_hbm, w2_hbm)
        acols = (a0_hbm, a1_hbm, a2_hbm)
        xyws = (xyw0, xyw1, xyw2)
        xyas = (xya0, xya1, xya2)
        yzws = (yzw0, yzw1, yzw2)
        yzas = (yza0, yza1, yza2)
        xzs = (xz0, xz1, xz2)

        ds0 = [
            pltpu.async_copy(winv_hbm.at[osl], winv, sem1),
            pltpu.async_copy(gxy_hbm.at[osl], rxy, sem1),
            pltpu.async_copy(gyz_hbm.at[osl], ryz, sem1),
            pltpu.async_copy(wsp_hbm, wv, sem1),
        ]
        for c in range(3):
            ds0.append(pltpu.async_copy(acols[c].at[pl.ds(base, RPW)], xzs[c], sem1))
        for d in ds0:
            d.wait()

        lanes0 = lax.iota(jnp.int32, L)

        def clamp_body(i, _):
            csl = pl.ds(i * L, L)
            pos = i * L + lanes0
            vxy = rxy[csl]
            vyz = ryz[csl]
            xywi[csl] = jnp.where(vxy < M, vxy, pos)
            xyai[csl] = jnp.where(vxy >= M, vxy - M, pos)
            yzwi[csl] = jnp.where(vyz < M, vyz, pos)
            yzai[csl] = jnp.where(vyz >= M, vyz - M, pos)
            return 0

        lax.fori_loop(0, RPW // L, clamp_body, 0)

        ds1 = []
        for j in range(NCH):
            sl = pl.ds(j * 128, 128)
            for c in range(3):
                ds1.append(pltpu.async_copy(acols[c].at[xyai.at[sl]],
                                            xyas[c].at[sl], sem2))
                ds1.append(pltpu.async_copy(acols[c].at[yzai.at[sl]],
                                            yzas[c].at[sl], sem2))
        for d in ds1:
            d.wait()

        wvec = wv[...]
        ws = [wvec[k] for k in range(N_KENN_LAYERS * 3)]
        lanes = lax.iota(jnp.int32, L)

        def compute_body(i, _):
            sl = pl.ds(i * L, L)
            rows16 = i * L + lanes
            has = winv[sl] >= 0
            xy_is_a = rxy[sl] >= M
            yz_is_a = ryz[sl] >= M
            xy = [jnp.where(xy_is_a, xyas[c][sl], xyws[c][sl]) for c in range(3)]
            yz = [jnp.where(yz_is_a, yzas[c][sl], yzws[c][sl]) for c in range(3)]
            xz = [xzs[c][sl] for c in range(3)]
            xz_orig = list(xz)
            for l in range(N_KENN_LAYERS):
                for c in range(3):
                    a, b, dd = -xy[c], -yz[c], xz[c]
                    w = ws[l * 3 + c]
                    win0 = (a >= b) & (a >= dd)
                    win1 = (~win0) & (b >= dd)
                    win2 = ~(win0 | win1)
                    xy[c] = xy[c] - jnp.where(win0, w, 0.0)
                    yz[c] = yz[c] - jnp.where(win1, w, 0.0)
                    xz[c] = xz[c] + jnp.where(win2, w, 0.0)
            o = [jnp.where(has, xz[c], xz_orig[c]) for c in range(3)]
            m = jnp.maximum(jnp.maximum(o[0], o[1]), o[2])
            ex = [jnp.exp(o[c] - m) for c in range(3)]
            ssum = ex[0] + ex[1] + ex[2]
            for c in range(3):
                cvec = jnp.full((L,), c, jnp.int32)
                plsc.store_scatter(obf, [rows16, cvec], o[c])
                plsc.store_scatter(sbf, [rows16, cvec], ex[c] / ssum)
            return 0

        lax.fori_loop(0, RPW // L, compute_body, 0)

        pltpu.sync_copy(obf, out_hbm.at[pl.ds(base, RPW), :])
        pltpu.sync_copy(sbf, soft_hbm.at[pl.ds(base, RPW), :])

    return b2_kernel


def kernel(features, within_preactivations, index_xy, index_yz, index_xz,
           W1, b1, W2, b2, clause_weights):
    M = within_preactivations.shape[0]
    N = features.shape[0]
    E = index_xz.shape[0]

    a0, a1, a2 = _mlp_cols(features, W1, b1, W2, b2)
    w0 = within_preactivations[:, 0]
    w1 = within_preactivations[:, 1]
    w2 = within_preactivations[:, 2]

    wsp = jax.nn.softplus(clause_weights).reshape(-1)
    wsp16 = jnp.zeros((L,), jnp.float32).at[: wsp.shape[0]].set(wsp)

    winners = _make_winner_kernel(E, M, N)(index_xz)
    winv, gxy, gyz = _make_b1_kernel(E, M, N)(
        winners, index_xy, index_yz
    )
    out, soft = _make_b2_kernel(E, M, N)(
        winv, gxy, gyz, w0, w1, w2, a0, a1, a2, wsp16
    )
    return (out, soft)


# speedup vs baseline: 1.6695x; 1.6695x over previous
import functools

import jax
import jax.numpy as jnp
from jax import lax
from jax.experimental import pallas as pl
from jax.experimental.pallas import tpu as pltpu
from jax.experimental.pallas import tpu_sc as plsc

N_KENN_LAYERS = 3
NC = 2
NS = 16
NW = NC * NS
L = 16

RPW = 384
NCH = RPW // 128


def _mlp_body(x_ref, w1_ref, b1_ref, w2_ref, b2_ref, o0_ref, o1_ref, o2_ref):
    h = jnp.maximum(
        jnp.dot(x_ref[...], w1_ref[...], preferred_element_type=jnp.float32)
        + b1_ref[...],
        0.0,
    )
    res = jax.lax.dot_general(
        w2_ref[...], h, (((0,), (1,)), ((), ())),
        preferred_element_type=jnp.float32,
    ) + b2_ref[...]
    o0_ref[...] = res[0]
    o1_ref[...] = res[1]
    o2_ref[...] = res[2]


def _mlp_cols(features, W1, b1, W2, b2):
    n, d = features.shape
    blk = 1024
    grid = (n + blk - 1) // blk
    col = jax.ShapeDtypeStruct((n,), jnp.float32)
    return pl.pallas_call(
        _mlp_body,
        grid=(grid,),
        in_specs=[
            pl.BlockSpec((blk, d), lambda i: (i, 0)),
            pl.BlockSpec((d, W1.shape[1]), lambda i: (0, 0)),
            pl.BlockSpec((1, W1.shape[1]), lambda i: (0, 0)),
            pl.BlockSpec((W1.shape[1], W2.shape[1]), lambda i: (0, 0)),
            pl.BlockSpec((W2.shape[1], 1), lambda i: (0, 0)),
        ],
        out_specs=[
            pl.BlockSpec((blk,), lambda i: (i,)),
            pl.BlockSpec((blk,), lambda i: (i,)),
            pl.BlockSpec((blk,), lambda i: (i,)),
        ],
        out_shape=[col, col, col],
        compiler_params=pltpu.CompilerParams(
            dimension_semantics=("arbitrary",),
        ),
    )(features, W1, b1.reshape(1, -1), W2, b2.reshape(-1, 1))


def _make_winner_kernel(E, M, N):
    EPW = E // NW
    mesh = plsc.VectorSubcoreMesh(
        core_axis_name="c", subcore_axis_name="s", num_cores=NC, num_subcores=NS
    )

    @functools.partial(
        pl.kernel,
        out_type=jax.ShapeDtypeStruct((NW * N,), jnp.int32),
        mesh=mesh,
        compiler_params=pltpu.CompilerParams(needs_layout_passes=False),
        scratch_types=[
            pltpu.VMEM((EPW,), jnp.int32),
            pltpu.VMEM((N,), jnp.int32),
            pltpu.VMEM((EPW + L,), jnp.int32),
            pltpu.VMEM((EPW + L,), jnp.int32),
            pltpu.SemaphoreType.DMA,
        ],
    )
    def winner_kernel(idxxz_hbm, win_hbm, idx_v, win_v, rowc, evc, sem):
        wid = lax.axis_index("s") * NC + lax.axis_index("c")
        pltpu.async_copy(idxxz_hbm.at[pl.ds(wid * EPW, EPW)], idx_v, sem).wait()

        neg1 = jnp.full((L,), -1, jnp.int32)

        def init_body(i, _):
            win_v[pl.ds(i * L, L)] = neg1
            return 0

        lax.fori_loop(0, N // L, init_body, 0)

        lanes = lax.iota(jnp.int32, L)
        ebase = wid * EPW

        def compact_body(i, off):
            v = idx_v[pl.ds(i * L, L)]
            msk = v >= M
            evec = (ebase + i * L) + lanes
            plsc.store_compressed(rowc.at[pl.ds(off, L)], v - M, mask=msk)
            plsc.store_compressed(evc.at[pl.ds(off, L)], evec, mask=msk)
            return off + plsc.all_reduce_population_count(msk)[0]

        total = lax.fori_loop(0, EPW // L, compact_body, 0)

        def settle_body(j, _):
            rows = rowc[pl.ds(j * L, L)]
            evec = evc[pl.ds(j * L, L)]
            lanemask = (j * L + lanes) < total

            def w_cond(b):
                return jnp.max(b) > 0

            def w_body(b):
                plsc.store_scatter(win_v, [rows], evec, mask=b != 0)
                g = plsc.load_gather(win_v, [rows], mask=lanemask)
                return (lanemask & (g < evec)).astype(jnp.int32)

            lax.while_loop(w_cond, w_body, lanemask.astype(jnp.int32))
            return 0

        lax.fori_loop(0, (total + L - 1) // L, settle_body, 0)
        pltpu.sync_copy(win_v, win_hbm.at[pl.ds(wid * N, N)])

    return winner_kernel


def _make_b1_kernel(E, M, N):
    mesh = plsc.VectorSubcoreMesh(
        core_axis_name="c", subcore_axis_name="s", num_cores=NC, num_subcores=NS
    )
    flat = jax.ShapeDtypeStruct((NW * RPW,), jnp.int32)

    @functools.partial(
        pl.kernel,
        out_type=(flat, flat, flat),
        mesh=mesh,
        compiler_params=pltpu.CompilerParams(needs_layout_passes=False),
        scratch_types=[
            pltpu.VMEM((NW * RPW,), jnp.int32),
            pltpu.VMEM((RPW,), jnp.int32),
            pltpu.VMEM((RPW,), jnp.int32),
            pltpu.VMEM((RPW,), jnp.int32),
            pltpu.VMEM((RPW,), jnp.int32),
            pltpu.SemaphoreType.DMA,
        ],
    )
    def b1_kernel(win_hbm, ixy_hbm, iyz_hbm,
                  winv_hbm, gxy_hbm, gyz_hbm,
                  winb, winv, ebuf, gxy, gyz, sem):
        wid = lax.axis_index("s") * NC + lax.axis_index("c")
        base = jnp.minimum(wid * RPW, N - RPW)
        obase = wid * RPW

        ds0 = [
            pltpu.async_copy(win_hbm.at[pl.ds(t * N + base, RPW)],
                             winb.at[pl.ds(t * RPW, RPW)], sem)
            for t in range(NW)
        ]
        for d in ds0:
            d.wait()

        def red_body(i, _):
            acc = jnp.full((L,), -1, jnp.int32)
            for t in range(NW):
                acc = jnp.maximum(acc, winb[pl.ds(t * RPW + i * L, L)])
            winv[pl.ds(i * L, L)] = acc
            ebuf[pl.ds(i * L, L)] = jnp.maximum(acc, 0)
            return 0

        lax.fori_loop(0, RPW // L, red_body, 0)

        ds1 = []
        for j in range(NCH):
            sl = pl.ds(j * 128, 128)
            ds1.append(pltpu.async_copy(ixy_hbm.at[ebuf.at[sl]], gxy.at[sl], sem))
            ds1.append(pltpu.async_copy(iyz_hbm.at[ebuf.at[sl]], gyz.at[sl], sem))
        for d in ds1:
            d.wait()

        osl = pl.ds(obase, RPW)
        pltpu.sync_copy(winv, winv_hbm.at[osl])
        pltpu.sync_copy(gxy, gxy_hbm.at[osl])
        pltpu.sync_copy(gyz, gyz_hbm.at[osl])

    return b1_kernel


def _make_b2_kernel(E, M, N):
    mesh = plsc.VectorSubcoreMesh(
        core_axis_name="c", subcore_axis_name="s", num_cores=NC, num_subcores=NS
    )

    @functools.partial(
        pl.kernel,
        out_type=(
            jax.ShapeDtypeStruct((N, 3), jnp.float32),
            jax.ShapeDtypeStruct((N, 3), jnp.float32),
        ),
        mesh=mesh,
        compiler_params=pltpu.CompilerParams(needs_layout_passes=False),
        scratch_types=[
            pltpu.VMEM((RPW,), jnp.int32),
            pltpu.VMEM((RPW,), jnp.int32),
            pltpu.VMEM((RPW,), jnp.int32),
            pltpu.VMEM((RPW,), jnp.int32),
            pltpu.VMEM((RPW,), jnp.int32),
            pltpu.VMEM((RPW,), jnp.int32),
            pltpu.VMEM((RPW,), jnp.int32),
            pltpu.VMEM((RPW,), jnp.float32),
            pltpu.VMEM((RPW,), jnp.float32),
            pltpu.VMEM((RPW,), jnp.float32),
            pltpu.VMEM((RPW,), jnp.float32),
            pltpu.VMEM((RPW,), jnp.float32),
            pltpu.VMEM((RPW,), jnp.float32),
            pltpu.VMEM((RPW,), jnp.float32),
            pltpu.VMEM((RPW,), jnp.float32),
            pltpu.VMEM((RPW,), jnp.float32),
            pltpu.VMEM((RPW,), jnp.float32),
            pltpu.VMEM((RPW,), jnp.float32),
            pltpu.VMEM((RPW,), jnp.float32),
            pltpu.VMEM((RPW,), jnp.float32),
            pltpu.VMEM((RPW,), jnp.float32),
            pltpu.VMEM((RPW,), jnp.float32),
            pltpu.VMEM((RPW, 3), jnp.float32),
            pltpu.VMEM((RPW, 3), jnp.float32),
            pltpu.VMEM((L,), jnp.float32),
            pltpu.SemaphoreType.DMA,
            pltpu.SemaphoreType.DMA,
        ],
    )
    def b2_kernel(winv_hbm, gxy_hbm, gyz_hbm,
                  w0_hbm, w1_hbm, w2_hbm, a0_hbm, a1_hbm, a2_hbm, wsp_hbm,
                  out_hbm, soft_hbm,
                  winv, rxy, ryz, xywi, xyai, yzwi, yzai,
                  xyw0, xyw1, xyw2, xya0, xya1, xya2,
                  yzw0, yzw1, yzw2, yza0, yza1, yza2,
                  xz0, xz1, xz2, obf, sbf, wv, sem1, sem2):
        wid = lax.axis_index("s") * NC + lax.axis_index("c")
        base = jnp.minimum(wid * RPW, N - RPW)
        obase = wid * RPW
        osl = pl.ds(obase, RPW)
        wcols = (w0_hbm, w1_hbm, w2_hbm)
        acols = (a0_hbm, a1_hbm, a2_hbm)
        xyws = (xyw0, xyw1, xyw2)
        xyas = (xya0, xya1, xya2)
        yzws = (yzw0, yzw1, yzw2)
        yzas = (yza0, yza1, yza2)
        xzs = (xz0, xz1, xz2)

        ds0 = [
            pltpu.async_copy(winv_hbm.at[osl], winv, sem1),
            pltpu.async_copy(gxy_hbm.at[osl], rxy, sem1),
            pltpu.async_copy(gyz_hbm.at[osl], ryz, sem1),
            pltpu.async_copy(wsp_hbm, wv, sem1),
        ]
        for c in range(3):
            ds0.append(pltpu.async_copy(acols[c].at[pl.ds(base, RPW)], xzs[c], sem1))
        for d in ds0:
            d.wait()

        lanes0 = lax.iota(jnp.int32, L)

        def clamp_body(i, _):
            csl = pl.ds(i * L, L)
            pos = base + i * L + lanes0
            vxy = rxy[csl]
            vyz = ryz[csl]
            xywi[csl] = jnp.where(vxy < M, vxy, pos)
            xyai[csl] = jnp.where(vxy >= M, vxy - M, pos)
            yzwi[csl] = jnp.where(vyz < M, vyz, pos)
            yzai[csl] = jnp.where(vyz >= M, vyz - M, pos)
            return 0

        lax.fori_loop(0, RPW // L, clamp_body, 0)

        ds1 = []
        for j in range(NCH):
            sl = pl.ds(j * 128, 128)
            for c in range(3):
                ds1.append(pltpu.async_copy(acols[c].at[xyai.at[sl]],
                                            xyas[c].at[sl], sem2))
                ds1.append(pltpu.async_copy(acols[c].at[yzai.at[sl]],
                                            yzas[c].at[sl], sem2))
        for d in ds1:
            d.wait()

        wvec = wv[...]
        ws = [wvec[k] for k in range(N_KENN_LAYERS * 3)]
        lanes = lax.iota(jnp.int32, L)

        def compute_body(i, _):
            sl = pl.ds(i * L, L)
            rows16 = i * L + lanes
            has = winv[sl] >= 0
            xy_is_a = rxy[sl] >= M
            yz_is_a = ryz[sl] >= M
            xy = [jnp.where(xy_is_a, xyas[c][sl], xyws[c][sl]) for c in range(3)]
            yz = [jnp.where(yz_is_a, yzas[c][sl], yzws[c][sl]) for c in range(3)]
            xz = [xzs[c][sl] for c in range(3)]
            xz_orig = list(xz)
            for l in range(N_KENN_LAYERS):
                for c in range(3):
                    a, b, dd = -xy[c], -yz[c], xz[c]
                    w = ws[l * 3 + c]
                    win0 = (a >= b) & (a >= dd)
                    win1 = (~win0) & (b >= dd)
                    win2 = ~(win0 | win1)
                    xy[c] = xy[c] - jnp.where(win0, w, 0.0)
                    yz[c] = yz[c] - jnp.where(win1, w, 0.0)
                    xz[c] = xz[c] + jnp.where(win2, w, 0.0)
            o = [jnp.where(has, xz[c], xz_orig[c]) for c in range(3)]
            m = jnp.maximum(jnp.maximum(o[0], o[1]), o[2])
            ex = [jnp.exp(o[c] - m) for c in range(3)]
            ssum = ex[0] + ex[1] + ex[2]
            for c in range(3):
                cvec = jnp.full((L,), c, jnp.int32)
                plsc.store_scatter(obf, [rows16, cvec], o[c])
                plsc.store_scatter(sbf, [rows16, cvec], ex[c] / ssum)
            return 0

        lax.fori_loop(0, RPW // L, compute_body, 0)

        pltpu.sync_copy(obf, out_hbm.at[pl.ds(base, RPW), :])
        pltpu.sync_copy(sbf, soft_hbm.at[pl.ds(base, RPW), :])

    return b2_kernel


def kernel(features, within_preactivations, index_xy, index_yz, index_xz,
           W1, b1, W2, b2, clause_weights):
    M = within_preactivations.shape[0]
    N = features.shape[0]
    E = index_xz.shape[0]

    a0, a1, a2 = _mlp_cols(features, W1, b1, W2, b2)
    w0 = within_preactivations[:, 0]
    w1 = within_preactivations[:, 1]
    w2 = within_preactivations[:, 2]

    wsp = jax.nn.softplus(clause_weights).reshape(-1)
    wsp16 = jnp.zeros((L,), jnp.float32).at[: wsp.shape[0]].set(wsp)

    winners = _make_winner_kernel(E, M, N)(index_xz)
    winv, gxy, gyz = _make_b1_kernel(E, M, N)(
        winners, index_xy, index_yz
    )
    out, soft = _make_b2_kernel(E, M, N)(
        winv, gxy, gyz, w0, w1, w2, a0, a1, a2, wsp16
    )
    return (out, soft)


# speedup vs baseline: 1.9484x; 1.1671x over previous
import functools

import jax
import jax.numpy as jnp
from jax import lax
from jax.experimental import pallas as pl
from jax.experimental.pallas import tpu as pltpu
from jax.experimental.pallas import tpu_sc as plsc

N_KENN_LAYERS = 3
NC = 2
NS = 16
NW = NC * NS
L = 16

RPW = 384
NCH = RPW // 128


def _mlp_body(x_ref, w1_ref, b1_ref, w2_ref, b2_ref, o0_ref, o1_ref, o2_ref):
    h = jnp.maximum(
        jnp.dot(x_ref[...], w1_ref[...], preferred_element_type=jnp.float32)
        + b1_ref[...],
        0.0,
    )
    res = jax.lax.dot_general(
        w2_ref[...], h, (((0,), (1,)), ((), ())),
        preferred_element_type=jnp.float32,
    ) + b2_ref[...]
    o0_ref[...] = res[0]
    o1_ref[...] = res[1]
    o2_ref[...] = res[2]


def _mlp_cols(features, W1, b1, W2, b2):
    n, d = features.shape
    blk = 1024
    grid = (n + blk - 1) // blk
    col = jax.ShapeDtypeStruct((n,), jnp.float32)
    return pl.pallas_call(
        _mlp_body,
        grid=(grid,),
        in_specs=[
            pl.BlockSpec((blk, d), lambda i: (i, 0)),
            pl.BlockSpec((d, W1.shape[1]), lambda i: (0, 0)),
            pl.BlockSpec((1, W1.shape[1]), lambda i: (0, 0)),
            pl.BlockSpec((W1.shape[1], W2.shape[1]), lambda i: (0, 0)),
            pl.BlockSpec((W2.shape[1], 1), lambda i: (0, 0)),
        ],
        out_specs=[
            pl.BlockSpec((blk,), lambda i: (i,)),
            pl.BlockSpec((blk,), lambda i: (i,)),
            pl.BlockSpec((blk,), lambda i: (i,)),
        ],
        out_shape=[col, col, col],
        compiler_params=pltpu.CompilerParams(
            dimension_semantics=("arbitrary",),
        ),
    )(features, W1, b1.reshape(1, -1), W2, b2.reshape(-1, 1))


def _make_winner_kernel(E, M, N):
    EPW = E // NW
    mesh = plsc.VectorSubcoreMesh(
        core_axis_name="c", subcore_axis_name="s", num_cores=NC, num_subcores=NS
    )

    @functools.partial(
        pl.kernel,
        out_type=jax.ShapeDtypeStruct((NW * N,), jnp.int32),
        mesh=mesh,
        compiler_params=pltpu.CompilerParams(needs_layout_passes=False),
        scratch_types=[
            pltpu.VMEM((EPW,), jnp.int32),
            pltpu.VMEM((N,), jnp.int32),
            pltpu.VMEM((EPW + L,), jnp.int32),
            pltpu.VMEM((EPW + L,), jnp.int32),
            pltpu.SemaphoreType.DMA,
        ],
    )
    def winner_kernel(idxxz_hbm, win_hbm, idx_v, win_v, rowc, evc, sem):
        wid = lax.axis_index("s") * NC + lax.axis_index("c")
        pltpu.async_copy(idxxz_hbm.at[pl.ds(wid * EPW, EPW)], idx_v, sem).wait()

        neg1 = jnp.full((L,), -1, jnp.int32)

        def init_body(i, _):
            win_v[pl.ds(i * L, L)] = neg1
            return 0

        lax.fori_loop(0, N // L, init_body, 0)

        lanes = lax.iota(jnp.int32, L)
        ebase = wid * EPW

        def compact_body(i, off):
            v = idx_v[pl.ds(i * L, L)]
            msk = v >= M
            evec = (ebase + i * L) + lanes
            plsc.store_compressed(rowc.at[pl.ds(off, L)], v - M, mask=msk)
            plsc.store_compressed(evc.at[pl.ds(off, L)], evec, mask=msk)
            return off + plsc.all_reduce_population_count(msk)[0]

        total = lax.fori_loop(0, EPW // L, compact_body, 0)

        def settle_body(j, _):
            rows = rowc[pl.ds(j * L, L)]
            evec = evc[pl.ds(j * L, L)]
            lanemask = (j * L + lanes) < total

            def w_cond(b):
                return jnp.max(b) > 0

            def w_body(b):
                plsc.store_scatter(win_v, [rows], evec, mask=b != 0)
                g = plsc.load_gather(win_v, [rows], mask=lanemask)
                return (lanemask & (g < evec)).astype(jnp.int32)

            lax.while_loop(w_cond, w_body, lanemask.astype(jnp.int32))
            return 0

        lax.fori_loop(0, (total + L - 1) // L, settle_body, 0)
        pltpu.sync_copy(win_v, win_hbm.at[pl.ds(wid * N, N)])

    return winner_kernel


def _make_b1_kernel(E, M, N):
    mesh = plsc.VectorSubcoreMesh(
        core_axis_name="c", subcore_axis_name="s", num_cores=NC, num_subcores=NS
    )
    flat = jax.ShapeDtypeStruct((NW * RPW,), jnp.int32)

    @functools.partial(
        pl.kernel,
        out_type=(flat, flat, flat),
        mesh=mesh,
        compiler_params=pltpu.CompilerParams(needs_layout_passes=False),
        scratch_types=[
            pltpu.VMEM((NW * RPW,), jnp.int32),
            pltpu.VMEM((RPW,), jnp.int32),
            pltpu.VMEM((RPW,), jnp.int32),
            pltpu.VMEM((RPW,), jnp.int32),
            pltpu.VMEM((RPW,), jnp.int32),
            pltpu.SemaphoreType.DMA,
        ],
    )
    def b1_kernel(win_hbm, ixy_hbm, iyz_hbm,
                  winv_hbm, gxy_hbm, gyz_hbm,
                  winb, winv, ebuf, gxy, gyz, sem):
        wid = lax.axis_index("s") * NC + lax.axis_index("c")
        base = jnp.minimum(wid * RPW, N - RPW)
        obase = wid * RPW

        ds0 = [
            pltpu.async_copy(win_hbm.at[pl.ds(t * N + base, RPW)],
                             winb.at[pl.ds(t * RPW, RPW)], sem)
            for t in range(NW)
        ]
        for d in ds0:
            d.wait()

        def red_body(i, _):
            acc = jnp.full((L,), -1, jnp.int32)
            for t in range(NW):
                acc = jnp.maximum(acc, winb[pl.ds(t * RPW + i * L, L)])
            winv[pl.ds(i * L, L)] = acc
            ebuf[pl.ds(i * L, L)] = jnp.maximum(acc, 0)
            return 0

        lax.fori_loop(0, RPW // L, red_body, 0)

        ds1 = []
        for j in range(NCH):
            sl = pl.ds(j * 128, 128)
            ds1.append(pltpu.async_copy(ixy_hbm.at[ebuf.at[sl]], gxy.at[sl], sem))
            ds1.append(pltpu.async_copy(iyz_hbm.at[ebuf.at[sl]], gyz.at[sl], sem))
        for d in ds1:
            d.wait()

        osl = pl.ds(obase, RPW)
        pltpu.sync_copy(winv, winv_hbm.at[osl])
        pltpu.sync_copy(gxy, gxy_hbm.at[osl])
        pltpu.sync_copy(gyz, gyz_hbm.at[osl])

    return b1_kernel


def _make_b2_kernel(E, M, N):
    mesh = plsc.VectorSubcoreMesh(
        core_axis_name="c", subcore_axis_name="s", num_cores=NC, num_subcores=NS
    )

    @functools.partial(
        pl.kernel,
        out_type=(
            jax.ShapeDtypeStruct((N, 3), jnp.float32),
            jax.ShapeDtypeStruct((N, 3), jnp.float32),
        ),
        mesh=mesh,
        compiler_params=pltpu.CompilerParams(needs_layout_passes=False),
        scratch_types=[
            pltpu.VMEM((RPW,), jnp.int32),
            pltpu.VMEM((RPW,), jnp.int32),
            pltpu.VMEM((RPW,), jnp.int32),
            pltpu.VMEM((RPW,), jnp.float32),
            pltpu.VMEM((RPW,), jnp.float32),
            pltpu.VMEM((RPW,), jnp.float32),
            pltpu.VMEM((RPW,), jnp.float32),
            pltpu.VMEM((RPW,), jnp.float32),
            pltpu.VMEM((RPW,), jnp.float32),
            pltpu.VMEM((RPW,), jnp.float32),
            pltpu.VMEM((RPW,), jnp.float32),
            pltpu.VMEM((RPW,), jnp.float32),
            pltpu.VMEM((RPW, 3), jnp.float32),
            pltpu.VMEM((RPW, 3), jnp.float32),
            pltpu.VMEM((L,), jnp.float32),
            pltpu.SemaphoreType.DMA,
            pltpu.SemaphoreType.DMA,
        ],
    )
    def b2_kernel(winv_hbm, gxy_hbm, gyz_hbm, p0_hbm, p1_hbm, p2_hbm, wsp_hbm,
                  out_hbm, soft_hbm,
                  winv, gxy, gyz,
                  xy0, xy1, xy2, yz0, yz1, yz2, xz0, xz1, xz2,
                  obf, sbf, wv, sem1, sem2):
        wid = lax.axis_index("s") * NC + lax.axis_index("c")
        base = jnp.minimum(wid * RPW, N - RPW)
        obase = wid * RPW
        osl = pl.ds(obase, RPW)
        pcols = (p0_hbm, p1_hbm, p2_hbm)
        xys = (xy0, xy1, xy2)
        yzs = (yz0, yz1, yz2)
        xzs = (xz0, xz1, xz2)

        ds0 = [
            pltpu.async_copy(winv_hbm.at[osl], winv, sem1),
            pltpu.async_copy(gxy_hbm.at[osl], gxy, sem1),
            pltpu.async_copy(gyz_hbm.at[osl], gyz, sem1),
            pltpu.async_copy(wsp_hbm, wv, sem1),
        ]
        for c in range(3):
            ds0.append(pltpu.async_copy(pcols[c].at[pl.ds(M + base, RPW)],
                                        xzs[c], sem1))
        for d in ds0:
            d.wait()

        ds1 = []
        for j in range(NCH):
            sl = pl.ds(j * 128, 128)
            for c in range(3):
                ds1.append(pltpu.async_copy(pcols[c].at[gxy.at[sl]],
                                            xys[c].at[sl], sem2))
                ds1.append(pltpu.async_copy(pcols[c].at[gyz.at[sl]],
                                            yzs[c].at[sl], sem2))
        for d in ds1:
            d.wait()

        wvec = wv[...]
        ws = [wvec[k] for k in range(N_KENN_LAYERS * 3)]
        lanes = lax.iota(jnp.int32, L)

        def compute_body(i, _):
            sl = pl.ds(i * L, L)
            rows16 = i * L + lanes
            has = winv[sl] >= 0
            xy = [xys[c][sl] for c in range(3)]
            yz = [yzs[c][sl] for c in range(3)]
            xz = [xzs[c][sl] for c in range(3)]
            xz_orig = list(xz)
            for l in range(N_KENN_LAYERS):
                for c in range(3):
                    a, b, dd = -xy[c], -yz[c], xz[c]
                    w = ws[l * 3 + c]
                    win0 = (a >= b) & (a >= dd)
                    win1 = (~win0) & (b >= dd)
                    win2 = ~(win0 | win1)
                    xy[c] = xy[c] - jnp.where(win0, w, 0.0)
                    yz[c] = yz[c] - jnp.where(win1, w, 0.0)
                    xz[c] = xz[c] + jnp.where(win2, w, 0.0)
            o = [jnp.where(has, xz[c], xz_orig[c]) for c in range(3)]
            m = jnp.maximum(jnp.maximum(o[0], o[1]), o[2])
            ex = [jnp.exp(o[c] - m) for c in range(3)]
            ssum = ex[0] + ex[1] + ex[2]
            for c in range(3):
                cvec = jnp.full((L,), c, jnp.int32)
                plsc.store_scatter(obf, [rows16, cvec], o[c])
                plsc.store_scatter(sbf, [rows16, cvec], ex[c] / ssum)
            return 0

        lax.fori_loop(0, RPW // L, compute_body, 0)

        pltpu.sync_copy(obf, out_hbm.at[pl.ds(base, RPW), :])
        pltpu.sync_copy(sbf, soft_hbm.at[pl.ds(base, RPW), :])

    return b2_kernel


def kernel(features, within_preactivations, index_xy, index_yz, index_xz,
           W1, b1, W2, b2, clause_weights):
    M = within_preactivations.shape[0]
    N = features.shape[0]
    E = index_xz.shape[0]

    a0, a1, a2 = _mlp_cols(features, W1, b1, W2, b2)
    p0 = jnp.concatenate([within_preactivations[:, 0], a0])
    p1 = jnp.concatenate([within_preactivations[:, 1], a1])
    p2 = jnp.concatenate([within_preactivations[:, 2], a2])

    wsp = jax.nn.softplus(clause_weights).reshape(-1)
    wsp16 = jnp.zeros((L,), jnp.float32).at[: wsp.shape[0]].set(wsp)

    winners = _make_winner_kernel(E, M, N)(index_xz)
    winv, gxy, gyz = _make_b1_kernel(E, M, N)(
        winners, index_xy, index_yz
    )
    out, soft = _make_b2_kernel(E, M, N)(
        winv, gxy, gyz, p0, p1, p2, wsp16
    )
    return (out, soft)


# speedup vs baseline: 2.2343x; 1.1467x over previous
import functools

import jax
import jax.numpy as jnp
from jax import lax
from jax.experimental import pallas as pl
from jax.experimental.pallas import tpu as pltpu
from jax.experimental.pallas import tpu_sc as plsc

N_KENN_LAYERS = 3
NC = 2
NS = 16
NW = NC * NS
L = 16

RPW = 384
NCH = RPW // 128


def _mlp_body(x_ref, w1_ref, b1_ref, w2_ref, b2_ref, o0_ref, o1_ref, o2_ref):
    h = jnp.maximum(
        jnp.dot(x_ref[...], w1_ref[...], preferred_element_type=jnp.float32)
        + b1_ref[...],
        0.0,
    )
    res = jax.lax.dot_general(
        w2_ref[...], h, (((0,), (1,)), ((), ())),
        preferred_element_type=jnp.float32,
    ) + b2_ref[...]
    o0_ref[...] = res[0]
    o1_ref[...] = res[1]
    o2_ref[...] = res[2]


def _mlp_cols(features, W1, b1, W2, b2):
    n, d = features.shape
    blk = 1024
    grid = (n + blk - 1) // blk
    col = jax.ShapeDtypeStruct((n,), jnp.float32)
    return pl.pallas_call(
        _mlp_body,
        grid=(grid,),
        in_specs=[
            pl.BlockSpec((blk, d), lambda i: (i, 0)),
            pl.BlockSpec((d, W1.shape[1]), lambda i: (0, 0)),
            pl.BlockSpec((1, W1.shape[1]), lambda i: (0, 0)),
            pl.BlockSpec((W1.shape[1], W2.shape[1]), lambda i: (0, 0)),
            pl.BlockSpec((W2.shape[1], 1), lambda i: (0, 0)),
        ],
        out_specs=[
            pl.BlockSpec((blk,), lambda i: (i,)),
            pl.BlockSpec((blk,), lambda i: (i,)),
            pl.BlockSpec((blk,), lambda i: (i,)),
        ],
        out_shape=[col, col, col],
        compiler_params=pltpu.CompilerParams(
            dimension_semantics=("arbitrary",),
        ),
    )(features, W1, b1.reshape(1, -1), W2, b2.reshape(-1, 1))


def _make_winner_kernel(E, M, N):
    EPW = E // NW
    mesh = plsc.VectorSubcoreMesh(
        core_axis_name="c", subcore_axis_name="s", num_cores=NC, num_subcores=NS
    )

    @functools.partial(
        pl.kernel,
        out_type=jax.ShapeDtypeStruct((NW * N,), jnp.int32),
        mesh=mesh,
        compiler_params=pltpu.CompilerParams(needs_layout_passes=False),
        scratch_types=[
            pltpu.VMEM((EPW,), jnp.int32),
            pltpu.VMEM((N,), jnp.int32),
            pltpu.VMEM((EPW + L,), jnp.int32),
            pltpu.VMEM((EPW + L,), jnp.int32),
            pltpu.SemaphoreType.DMA,
        ],
    )
    def winner_kernel(idxxz_hbm, win_hbm, idx_v, win_v, rowc, evc, sem):
        wid = lax.axis_index("s") * NC + lax.axis_index("c")
        pltpu.async_copy(idxxz_hbm.at[pl.ds(wid * EPW, EPW)], idx_v, sem).wait()

        neg1 = jnp.full((L,), -1, jnp.int32)

        def init_body(i, _):
            win_v[pl.ds(i * L, L)] = neg1
            return 0

        lax.fori_loop(0, N // L, init_body, 0)

        lanes = lax.iota(jnp.int32, L)
        ebase = wid * EPW

        def compact_body(i, off):
            v = idx_v[pl.ds(i * L, L)]
            msk = v >= M
            evec = (ebase + i * L) + lanes
            plsc.store_compressed(rowc.at[pl.ds(off, L)], v - M, mask=msk)
            plsc.store_compressed(evc.at[pl.ds(off, L)], evec, mask=msk)
            return off + plsc.all_reduce_population_count(msk)[0]

        total = lax.fori_loop(0, EPW // L, compact_body, 0)

        def settle_body(j, _):
            rows = rowc[pl.ds(j * L, L)]
            evec = evc[pl.ds(j * L, L)]
            lanemask = (j * L + lanes) < total

            def w_cond(b):
                return jnp.max(b) > 0

            def w_body(b):
                plsc.store_scatter(win_v, [rows], evec, mask=b != 0)
                g = plsc.load_gather(win_v, [rows], mask=lanemask)
                return (lanemask & (g < evec)).astype(jnp.int32)

            lax.while_loop(w_cond, w_body, lanemask.astype(jnp.int32))
            return 0

        lax.fori_loop(0, (total + L - 1) // L, settle_body, 0)
        pltpu.sync_copy(win_v, win_hbm.at[pl.ds(wid * N, N)])

    return winner_kernel


def _make_b1_kernel(E, M, N):
    mesh = plsc.VectorSubcoreMesh(
        core_axis_name="c", subcore_axis_name="s", num_cores=NC, num_subcores=NS
    )
    flat = jax.ShapeDtypeStruct((NW * RPW,), jnp.int32)

    @functools.partial(
        pl.kernel,
        out_type=(flat, flat, flat),
        mesh=mesh,
        compiler_params=pltpu.CompilerParams(needs_layout_passes=False),
        scratch_types=[
            pltpu.VMEM((NW * RPW,), jnp.int32),
            pltpu.VMEM((RPW,), jnp.int32),
            pltpu.VMEM((RPW,), jnp.int32),
            pltpu.VMEM((RPW,), jnp.int32),
            pltpu.VMEM((RPW,), jnp.int32),
            pltpu.SemaphoreType.DMA,
        ],
    )
    def b1_kernel(win_hbm, ixy_hbm, iyz_hbm, dep_hbm,
                  winv_hbm, gxy_hbm, gyz_hbm,
                  winb, winv, ebuf, gxy, gyz, sem):
        del dep_hbm
        wid = lax.axis_index("s") * NC + lax.axis_index("c")
        base = jnp.minimum(wid * RPW, N - RPW)
        obase = wid * RPW

        ds0 = [
            pltpu.async_copy(win_hbm.at[pl.ds(t * N + base, RPW)],
                             winb.at[pl.ds(t * RPW, RPW)], sem)
            for t in range(NW)
        ]
        for d in ds0:
            d.wait()

        def red_body(i, _):
            acc = jnp.full((L,), -1, jnp.int32)
            for t in range(NW):
                acc = jnp.maximum(acc, winb[pl.ds(t * RPW + i * L, L)])
            winv[pl.ds(i * L, L)] = acc
            ebuf[pl.ds(i * L, L)] = jnp.maximum(acc, 0)
            return 0

        lax.fori_loop(0, RPW // L, red_body, 0)

        ds1 = []
        for j in range(NCH):
            sl = pl.ds(j * 128, 128)
            ds1.append(pltpu.async_copy(ixy_hbm.at[ebuf.at[sl]], gxy.at[sl], sem))
            ds1.append(pltpu.async_copy(iyz_hbm.at[ebuf.at[sl]], gyz.at[sl], sem))
        for d in ds1:
            d.wait()

        osl = pl.ds(obase, RPW)
        pltpu.sync_copy(winv, winv_hbm.at[osl])
        pltpu.sync_copy(gxy, gxy_hbm.at[osl])
        pltpu.sync_copy(gyz, gyz_hbm.at[osl])

    return b1_kernel


def _make_b2_kernel(E, M, N):
    mesh = plsc.VectorSubcoreMesh(
        core_axis_name="c", subcore_axis_name="s", num_cores=NC, num_subcores=NS
    )

    @functools.partial(
        pl.kernel,
        out_type=(
            jax.ShapeDtypeStruct((N, 3), jnp.float32),
            jax.ShapeDtypeStruct((N, 3), jnp.float32),
        ),
        mesh=mesh,
        compiler_params=pltpu.CompilerParams(needs_layout_passes=False),
        scratch_types=[
            pltpu.VMEM((RPW,), jnp.int32),
            pltpu.VMEM((RPW,), jnp.int32),
            pltpu.VMEM((RPW,), jnp.int32),
            pltpu.VMEM((RPW,), jnp.float32),
            pltpu.VMEM((RPW,), jnp.float32),
            pltpu.VMEM((RPW,), jnp.float32),
            pltpu.VMEM((RPW,), jnp.float32),
            pltpu.VMEM((RPW,), jnp.float32),
            pltpu.VMEM((RPW,), jnp.float32),
            pltpu.VMEM((RPW,), jnp.float32),
            pltpu.VMEM((RPW,), jnp.float32),
            pltpu.VMEM((RPW,), jnp.float32),
            pltpu.VMEM((RPW, 3), jnp.float32),
            pltpu.VMEM((RPW, 3), jnp.float32),
            pltpu.VMEM((L,), jnp.float32),
            pltpu.SemaphoreType.DMA,
            pltpu.SemaphoreType.DMA,
        ],
    )
    def b2_kernel(winv_hbm, gxy_hbm, gyz_hbm, p0_hbm, p1_hbm, p2_hbm, wsp_hbm,
                  out_hbm, soft_hbm,
                  winv, gxy, gyz,
                  xy0, xy1, xy2, yz0, yz1, yz2, xz0, xz1, xz2,
                  obf, sbf, wv, sem1, sem2):
        wid = lax.axis_index("s") * NC + lax.axis_index("c")
        base = jnp.minimum(wid * RPW, N - RPW)
        obase = wid * RPW
        osl = pl.ds(obase, RPW)
        pcols = (p0_hbm, p1_hbm, p2_hbm)
        xys = (xy0, xy1, xy2)
        yzs = (yz0, yz1, yz2)
        xzs = (xz0, xz1, xz2)

        ds0 = [
            pltpu.async_copy(winv_hbm.at[osl], winv, sem1),
            pltpu.async_copy(gxy_hbm.at[osl], gxy, sem1),
            pltpu.async_copy(gyz_hbm.at[osl], gyz, sem1),
            pltpu.async_copy(wsp_hbm, wv, sem1),
        ]
        for c in range(3):
            ds0.append(pltpu.async_copy(pcols[c].at[pl.ds(M + base, RPW)],
                                        xzs[c], sem1))
        for d in ds0:
            d.wait()

        ds1 = []
        for j in range(NCH):
            sl = pl.ds(j * 128, 128)
            for c in range(3):
                ds1.append(pltpu.async_copy(pcols[c].at[gxy.at[sl]],
                                            xys[c].at[sl], sem2))
                ds1.append(pltpu.async_copy(pcols[c].at[gyz.at[sl]],
                                            yzs[c].at[sl], sem2))
        for d in ds1:
            d.wait()

        wvec = wv[...]
        ws = [wvec[k] for k in range(N_KENN_LAYERS * 3)]
        lanes = lax.iota(jnp.int32, L)

        def compute_body(i, _):
            sl = pl.ds(i * L, L)
            rows16 = i * L + lanes
            has = winv[sl] >= 0
            xy = [xys[c][sl] for c in range(3)]
            yz = [yzs[c][sl] for c in range(3)]
            xz = [xzs[c][sl] for c in range(3)]
            xz_orig = list(xz)
            for l in range(N_KENN_LAYERS):
                for c in range(3):
                    a, b, dd = -xy[c], -yz[c], xz[c]
                    w = ws[l * 3 + c]
                    win0 = (a >= b) & (a >= dd)
                    win1 = (~win0) & (b >= dd)
                    win2 = ~(win0 | win1)
                    xy[c] = xy[c] - jnp.where(win0, w, 0.0)
                    yz[c] = yz[c] - jnp.where(win1, w, 0.0)
                    xz[c] = xz[c] + jnp.where(win2, w, 0.0)
            o = [jnp.where(has, xz[c], xz_orig[c]) for c in range(3)]
            m = jnp.maximum(jnp.maximum(o[0], o[1]), o[2])
            ex = [jnp.exp(o[c] - m) for c in range(3)]
            ssum = ex[0] + ex[1] + ex[2]
            for c in range(3):
                cvec = jnp.full((L,), c, jnp.int32)
                plsc.store_scatter(obf, [rows16, cvec], o[c])
                plsc.store_scatter(sbf, [rows16, cvec], ex[c] / ssum)
            return 0

        lax.fori_loop(0, RPW // L, compute_body, 0)

        pltpu.sync_copy(obf, out_hbm.at[pl.ds(base, RPW), :])
        pltpu.sync_copy(sbf, soft_hbm.at[pl.ds(base, RPW), :])

    return b2_kernel


def kernel(features, within_preactivations, index_xy, index_yz, index_xz,
           W1, b1, W2, b2, clause_weights):
    M = within_preactivations.shape[0]
    N = features.shape[0]
    E = index_xz.shape[0]

    a0, a1, a2 = _mlp_cols(features, W1, b1, W2, b2)
    p0 = jnp.concatenate([within_preactivations[:, 0], a0])
    p1 = jnp.concatenate([within_preactivations[:, 1], a1])
    p2 = jnp.concatenate([within_preactivations[:, 2], a2])

    wsp = jax.nn.softplus(clause_weights).reshape(-1)
    wsp16 = jnp.zeros((L,), jnp.float32).at[: wsp.shape[0]].set(wsp)

    winners = _make_winner_kernel(E, M, N)(index_xz)
    winv, gxy, gyz = _make_b1_kernel(E, M, N)(
        winners, index_xy, index_yz, a0
    )
    out, soft = _make_b2_kernel(E, M, N)(
        winv, gxy, gyz, p0, p1, p2, wsp16
    )
    return (out, soft)


# speedup vs baseline: 2.3280x; 1.0420x over previous
import functools

import jax
import jax.numpy as jnp
from jax import lax
from jax.experimental import pallas as pl
from jax.experimental.pallas import tpu as pltpu
from jax.experimental.pallas import tpu_sc as plsc

N_KENN_LAYERS = 3
NC = 2
NS = 16
NW = NC * NS
L = 16

RPW = 384
NCH = RPW // 128


def _mlp_body(x_ref, w1_ref, b1_ref, w2_ref, b2_ref, o0_ref, o1_ref, o2_ref):
    h = jnp.maximum(
        jnp.dot(x_ref[...], w1_ref[...], preferred_element_type=jnp.float32)
        + b1_ref[...],
        0.0,
    )
    res = jax.lax.dot_general(
        w2_ref[...], h, (((0,), (1,)), ((), ())),
        preferred_element_type=jnp.float32,
    ) + b2_ref[...]
    o0_ref[...] = res[0]
    o1_ref[...] = res[1]
    o2_ref[...] = res[2]


def _mlp_cols(features, W1, b1, W2, b2):
    n, d = features.shape
    blk = 1024
    grid = (n + blk - 1) // blk
    col = jax.ShapeDtypeStruct((n,), jnp.float32)
    return pl.pallas_call(
        _mlp_body,
        grid=(grid,),
        in_specs=[
            pl.BlockSpec((blk, d), lambda i: (i, 0)),
            pl.BlockSpec((d, W1.shape[1]), lambda i: (0, 0)),
            pl.BlockSpec((1, W1.shape[1]), lambda i: (0, 0)),
            pl.BlockSpec((W1.shape[1], W2.shape[1]), lambda i: (0, 0)),
            pl.BlockSpec((W2.shape[1], 1), lambda i: (0, 0)),
        ],
        out_specs=[
            pl.BlockSpec((blk,), lambda i: (i,)),
            pl.BlockSpec((blk,), lambda i: (i,)),
            pl.BlockSpec((blk,), lambda i: (i,)),
        ],
        out_shape=[col, col, col],
        compiler_params=pltpu.CompilerParams(
            dimension_semantics=("arbitrary",),
        ),
    )(features, W1, b1.reshape(1, -1), W2, b2.reshape(-1, 1))


def _make_winner_kernel(E, M, N):
    EPW = E // NW
    mesh = plsc.VectorSubcoreMesh(
        core_axis_name="c", subcore_axis_name="s", num_cores=NC, num_subcores=NS
    )

    @functools.partial(
        pl.kernel,
        out_type=jax.ShapeDtypeStruct((NW * N,), jnp.int32),
        mesh=mesh,
        compiler_params=pltpu.CompilerParams(needs_layout_passes=False),
        scratch_types=[
            pltpu.VMEM((EPW,), jnp.int32),
            pltpu.VMEM((N,), jnp.int32),
            pltpu.VMEM((EPW + L,), jnp.int32),
            pltpu.VMEM((EPW + L,), jnp.int32),
            pltpu.SemaphoreType.DMA,
        ],
    )
    def winner_kernel(idxxz_hbm, win_hbm, idx_v, win_v, rowc, evc, sem):
        wid = lax.axis_index("s") * NC + lax.axis_index("c")
        pltpu.async_copy(idxxz_hbm.at[pl.ds(wid * EPW, EPW)], idx_v, sem).wait()

        neg1 = jnp.full((L,), -1, jnp.int32)

        def init_body(i, _):
            win_v[pl.ds(i * L, L)] = neg1
            return 0

        lax.fori_loop(0, N // L, init_body, 0)

        lanes = lax.iota(jnp.int32, L)
        ebase = wid * EPW

        def compact_body(i, off):
            v = idx_v[pl.ds(i * L, L)]
            msk = v >= M
            evec = (ebase + i * L) + lanes
            plsc.store_compressed(rowc.at[pl.ds(off, L)], v - M, mask=msk)
            plsc.store_compressed(evc.at[pl.ds(off, L)], evec, mask=msk)
            return off + plsc.all_reduce_population_count(msk)[0]

        total = lax.fori_loop(0, EPW // L, compact_body, 0)

        def settle_body(j, _):
            rows = rowc[pl.ds(j * L, L)]
            evec = evc[pl.ds(j * L, L)]
            lanemask = (j * L + lanes) < total

            def w_cond(b):
                return jnp.max(b) > 0

            def w_body(b):
                plsc.store_scatter(win_v, [rows], evec, mask=b != 0)
                g = plsc.load_gather(win_v, [rows], mask=lanemask)
                return (lanemask & (g < evec)).astype(jnp.int32)

            lax.while_loop(w_cond, w_body, lanemask.astype(jnp.int32))
            return 0

        lax.fori_loop(0, (total + L - 1) // L, settle_body, 0)
        pltpu.sync_copy(win_v, win_hbm.at[pl.ds(wid * N, N)])

    return winner_kernel


def _make_bmerged_kernel(E, M, N):
    mesh = plsc.VectorSubcoreMesh(
        core_axis_name="c", subcore_axis_name="s", num_cores=NC, num_subcores=NS
    )

    @functools.partial(
        pl.kernel,
        out_type=(
            jax.ShapeDtypeStruct((N, 3), jnp.float32),
            jax.ShapeDtypeStruct((N, 3), jnp.float32),
        ),
        mesh=mesh,
        compiler_params=pltpu.CompilerParams(needs_layout_passes=False),
        scratch_types=[
            pltpu.VMEM((NW * RPW,), jnp.int32),
            pltpu.VMEM((RPW,), jnp.int32),
            pltpu.VMEM((RPW,), jnp.int32),
            pltpu.VMEM((RPW,), jnp.int32),
            pltpu.VMEM((RPW,), jnp.int32),
            pltpu.VMEM((RPW,), jnp.float32),
            pltpu.VMEM((RPW,), jnp.float32),
            pltpu.VMEM((RPW,), jnp.float32),
            pltpu.VMEM((RPW,), jnp.float32),
            pltpu.VMEM((RPW,), jnp.float32),
            pltpu.VMEM((RPW,), jnp.float32),
            pltpu.VMEM((RPW,), jnp.float32),
            pltpu.VMEM((RPW,), jnp.float32),
            pltpu.VMEM((RPW,), jnp.float32),
            pltpu.VMEM((RPW, 3), jnp.float32),
            pltpu.VMEM((RPW, 3), jnp.float32),
            pltpu.VMEM((L,), jnp.float32),
            pltpu.SemaphoreType.DMA,
            pltpu.SemaphoreType.DMA,
            pltpu.SemaphoreType.DMA,
        ],
    )
    def b_kernel(win_hbm, ixy_hbm, iyz_hbm, p0_hbm, p1_hbm, p2_hbm, wsp_hbm,
                 out_hbm, soft_hbm,
                 winb, winv, ebuf, gxy, gyz,
                 xy0, xy1, xy2, yz0, yz1, yz2, xz0, xz1, xz2,
                 obf, sbf, wv, sem1, sem2, sem3):
        wid = lax.axis_index("s") * NC + lax.axis_index("c")
        base = jnp.minimum(wid * RPW, N - RPW)
        pcols = (p0_hbm, p1_hbm, p2_hbm)
        xys = (xy0, xy1, xy2)
        yzs = (yz0, yz1, yz2)
        xzs = (xz0, xz1, xz2)

        ds0 = [
            pltpu.async_copy(win_hbm.at[pl.ds(t * N + base, RPW)],
                             winb.at[pl.ds(t * RPW, RPW)], sem1)
            for t in range(NW)
        ]
        ds0.append(pltpu.async_copy(wsp_hbm, wv, sem1))
        for c in range(3):
            ds0.append(pltpu.async_copy(pcols[c].at[pl.ds(M + base, RPW)],
                                        xzs[c], sem1))
        for d in ds0:
            d.wait()

        def red_body(i, _):
            acc = jnp.full((L,), -1, jnp.int32)
            for t in range(NW):
                acc = jnp.maximum(acc, winb[pl.ds(t * RPW + i * L, L)])
            winv[pl.ds(i * L, L)] = acc
            ebuf[pl.ds(i * L, L)] = jnp.maximum(acc, 0)
            return 0

        lax.fori_loop(0, RPW // L, red_body, 0)

        ds1 = []
        for j in range(NCH):
            sl = pl.ds(j * 128, 128)
            ds1.append(pltpu.async_copy(ixy_hbm.at[ebuf.at[sl]], gxy.at[sl], sem2))
            ds1.append(pltpu.async_copy(iyz_hbm.at[ebuf.at[sl]], gyz.at[sl], sem2))
        for d in ds1:
            d.wait()

        chunk_descs = []
        for j in range(NCH):
            sl = pl.ds(j * 128, 128)
            cds = []
            for c in range(3):
                cds.append(pltpu.async_copy(pcols[c].at[gxy.at[sl]],
                                            xys[c].at[sl], sem3))
                cds.append(pltpu.async_copy(pcols[c].at[gyz.at[sl]],
                                            yzs[c].at[sl], sem3))
            chunk_descs.append(cds)

        wvec = wv[...]
        ws = [wvec[k] for k in range(N_KENN_LAYERS * 3)]
        lanes = lax.iota(jnp.int32, L)
        vec_per_chunk = 128 // L

        def compute_body(i, _):
            sl = pl.ds(i * L, L)
            rows16 = i * L + lanes
            has = winv[sl] >= 0
            xy = [xys[c][sl] for c in range(3)]
            yz = [yzs[c][sl] for c in range(3)]
            xz = [xzs[c][sl] for c in range(3)]
            xz_orig = list(xz)
            for l in range(N_KENN_LAYERS):
                for c in range(3):
                    a, b, dd = -xy[c], -yz[c], xz[c]
                    w = ws[l * 3 + c]
                    win0 = (a >= b) & (a >= dd)
                    win1 = (~win0) & (b >= dd)
                    win2 = ~(win0 | win1)
                    xy[c] = xy[c] - jnp.where(win0, w, 0.0)
                    yz[c] = yz[c] - jnp.where(win1, w, 0.0)
                    xz[c] = xz[c] + jnp.where(win2, w, 0.0)
            o = [jnp.where(has, xz[c], xz_orig[c]) for c in range(3)]
            m = jnp.maximum(jnp.maximum(o[0], o[1]), o[2])
            ex = [jnp.exp(o[c] - m) for c in range(3)]
            ssum = ex[0] + ex[1] + ex[2]
            for c in range(3):
                cvec = jnp.full((L,), c, jnp.int32)
                plsc.store_scatter(obf, [rows16, cvec], o[c])
                plsc.store_scatter(sbf, [rows16, cvec], ex[c] / ssum)
            return 0

        for j in range(NCH):
            for d in chunk_descs[j]:
                d.wait()
            lax.fori_loop(j * vec_per_chunk, (j + 1) * vec_per_chunk,
                          compute_body, 0)

        pltpu.sync_copy(obf, out_hbm.at[pl.ds(base, RPW), :])
        pltpu.sync_copy(sbf, soft_hbm.at[pl.ds(base, RPW), :])

    return b_kernel


def kernel(features, within_preactivations, index_xy, index_yz, index_xz,
           W1, b1, W2, b2, clause_weights):
    M = within_preactivations.shape[0]
    N = features.shape[0]
    E = index_xz.shape[0]

    a0, a1, a2 = _mlp_cols(features, W1, b1, W2, b2)
    p0 = jnp.concatenate([within_preactivations[:, 0], a0])
    p1 = jnp.concatenate([within_preactivations[:, 1], a1])
    p2 = jnp.concatenate([within_preactivations[:, 2], a2])

    wsp = jax.nn.softplus(clause_weights).reshape(-1)
    wsp16 = jnp.zeros((L,), jnp.float32).at[: wsp.shape[0]].set(wsp)

    winners = _make_winner_kernel(E, M, N)(index_xz)
    out, soft = _make_bmerged_kernel(E, M, N)(
        winners, index_xy, index_yz, p0, p1, p2, wsp16
    )
    return (out, soft)
